# baseline (speedup 1.0000x reference)
import math

import numpy as np
import jax
import jax.numpy as jnp
from jax.experimental import pallas as pl
from jax.experimental.pallas import tpu as pltpu

_T = 215
_TP = 216
_B = 128
_BB = 8
_R = _BB * _TP
_DINP = 36
_DM = 144
_DTR = 180
_NH = 4
_HD = 45
_DPE = 36
_NPE = _DPE // 2
_DFIN = 108

_TSCALES = (float(_T) ** np.linspace(0.0, 1.0, _NPE)).astype(np.float32)

_RSQ_D = 1.0 / math.sqrt(float(_DM))
_RSQ_HD = 1.0 / math.sqrt(float(_HD))
_SQRT_DM = math.sqrt(float(_DM))


def _dot(a, b):
    return jnp.dot(a, b, preferred_element_type=jnp.float32)


def _dotb(a, b):
    return jnp.dot(a.astype(jnp.bfloat16), b.astype(jnp.bfloat16),
                   preferred_element_type=jnp.float32)


def _bdot_qk(q, k):
    return jax.lax.dot_general(
        q, k, (((2,), (2,)), ((0,), (0,))),
        preferred_element_type=jnp.float32)


def _bdot_av(a, v):
    return jax.lax.dot_general(
        a, v, (((2,), (1,)), ((0,), (0,))),
        preferred_element_type=jnp.float32)


def _layer_norm(x, g, b):
    mu = jnp.mean(x, axis=-1, keepdims=True)
    var = jnp.mean((x - mu) ** 2, axis=-1, keepdims=True)
    return (x - mu) * jax.lax.rsqrt(var + 1e-5) * g + b


def _enc_layer(h, keepcol, Wqkv, bqkv, Wo, bo, W1, b1, W2, b2,
               g1, be1, g2, be2):
    qkv = _dot(h, Wqkv) + bqkv
    outs = []
    for hh in range(_NH):
        qh = qkv[:, hh * _HD:(hh + 1) * _HD].reshape(_BB, _TP, _HD)
        kh = qkv[:, _DTR + hh * _HD:_DTR + (hh + 1) * _HD].reshape(_BB, _TP, _HD)
        vh = qkv[:, 2 * _DTR + hh * _HD:2 * _DTR + (hh + 1) * _HD]
        s = _bdot_qk(qh, kh).reshape(_R, _TP)
        smax = jnp.max(s, axis=1, keepdims=True)
        p = jnp.exp(s - smax)
        vaug = jnp.concatenate([vh * keepcol, keepcol], axis=1)
        pv = _bdot_av(p.reshape(_BB, _TP, _TP),
                      vaug.reshape(_BB, _TP, _HD + 1)).reshape(_R, _HD + 1)
        outs.append(pv[:, :_HD] / pv[:, _HD:])
    o = jnp.concatenate(outs, axis=1)
    o = _dot(o, Wo) + bo
    h = _layer_norm(h + o, g1, be1)
    ff = jnp.maximum(_dot(h, W1) + b1, 0.0)
    ff = _dot(ff, W2) + b2
    return _layer_norm(h + ff, g2, be2)


def _fwd_kernel(src_ref, times_ref, len_ref, ts_ref,
                W_enc_ref, b_enc_ref,
                Wq_ref, bq_ref, Wk_ref, bk_ref, Wv_ref, bv_ref,
                Wskip_ref, bskip_ref,
                l0_Wqkv, l0_bqkv, l0_Wo, l0_bo, l0_W1, l0_b1, l0_W2, l0_b2,
                l0_g1, l0_be1, l0_g2, l0_be2,
                l1_Wqkv, l1_bqkv, l1_Wo, l1_bo, l1_W1, l1_b1, l1_W2, l1_b2,
                l1_g1, l1_be1, l1_g2, l1_be2,
                Wm1_ref, bm1_ref, Wm2_ref, bm2_ref,
                out_ref, alpha_ref):
    lens = jnp.concatenate(
        [jnp.broadcast_to(len_ref[j, 0, 0], (1, 1)) for j in range(_BB)],
        axis=0)

    xr = src_ref[:, :_DINP]
    x = (_dot(xr, W_enc_ref[...]) + b_enc_ref[...]) * _SQRT_DM

    sc = times_ref[...] * ts_ref[...]
    pe = jnp.concatenate([jnp.sin(sc), jnp.cos(sc)], axis=1)

    skip = _dot(x, Wskip_ref[...]) + bskip_ref[...]
    pieces = []
    for j in range(_BB):
        x36 = x[j * _TP:j * _TP + _DINP]
        q = _dot(x36, Wq_ref[...]) + bq_ref[...]
        k = _dot(x36, Wk_ref[...]) + bk_ref[...]
        v = _dot(x36, Wv_ref[...]) + bv_ref[...]
        s = _dot(q, k.T) * _RSQ_D
        smax = jnp.max(s, axis=1, keepdims=True)
        p = jnp.exp(s - smax)
        attn = p / (jnp.sum(p, axis=1, keepdims=True) + 1e-16)
        alpha_ref[j] = attn
        o_g = _dot(attn, v)
        pieces.append(skip[j * _TP:j * _TP + _DINP] + o_g)
        pieces.append(skip[j * _TP + _DINP:(j + 1) * _TP])
    outs = jnp.concatenate(pieces, axis=0)

    h = jnp.concatenate([outs, pe], axis=1)

    lens_rows = jnp.concatenate(
        [jnp.broadcast_to(lens[j, 0], (_TP, 1)) for j in range(_BB)], axis=0)
    t_col = jax.lax.rem(jax.lax.broadcasted_iota(jnp.int32, (_R, 1), 0),
                        jnp.int32(_TP))
    keepcol = (t_col < lens_rows).astype(jnp.float32)

    h = _enc_layer(h, keepcol,
                   l0_Wqkv[...], l0_bqkv[...], l0_Wo[...], l0_bo[...],
                   l0_W1[...], l0_b1[...], l0_W2[...], l0_b2[...],
                   l0_g1[...], l0_be1[...], l0_g2[...], l0_be2[...])
    h = _enc_layer(h, keepcol,
                   l1_Wqkv[...], l1_bqkv[...], l1_Wo[...], l1_bo[...],
                   l1_W1[...], l1_b1[...], l1_W2[...], l1_b2[...],
                   l1_g1[...], l1_be1[...], l1_g2[...], l1_be2[...])

    lane2 = jax.lax.broadcasted_iota(jnp.int32, (_BB, _R), 1)
    rowbase = jax.lax.broadcasted_iota(jnp.int32, (_BB, _R), 0) * _TP
    t_local = lane2 - rowbase
    keep = ((t_local >= 0) & (t_local < lens)).astype(jnp.float32)
    agg = _dot(keep, h) / (lens.astype(jnp.float32) + 1.0)

    feat = agg[:, :_DFIN]
    hid = jnp.maximum(_dot(feat, Wm1_ref[...]) + bm1_ref[...], 0.0)
    out_ref[...] = _dot(hid, Wm2_ref[...]) + bm2_ref[...]


def _dist_kernel(x_ref, o_ref):
    X = x_ref[...]

    def body(i, acc):
        row = x_ref[pl.ds(i, 1), :]
        d = X - row
        ssq = jnp.sum(d * d, axis=1, keepdims=True)
        return acc + jnp.sum(jnp.sqrt(jnp.maximum(ssq, 1e-24)))

    tot = jax.lax.fori_loop(0, _B, body, jnp.float32(0.0))
    o_ref[...] = jnp.broadcast_to(tot / float(_B * _B), (1, 1))


def _full2d(a):
    return pl.BlockSpec(a.shape, lambda b: (0,) * a.ndim)


def kernel(src, static, times, lengths, adj, W_enc, b_enc, W_emb, b_emb,
           Wq, bq, Wk, bk, Wv, bv, Wskip, bskip,
           l0_Wqkv, l0_bqkv, l0_Wo, l0_bo, l0_W1, l0_b1, l0_W2, l0_b2,
           l0_ln1_g, l0_ln1_b, l0_ln2_g, l0_ln2_b,
           l1_Wqkv, l1_bqkv, l1_Wo, l1_bo, l1_W1, l1_b1, l1_W2, l1_b2,
           l1_ln1_g, l1_ln1_b, l1_ln2_g, l1_ln2_b,
           Wm1, bm1, Wm2, bm2):
    f32 = jnp.float32
    def scale_qkv(W, b):
        Wd = jnp.concatenate([W[:, :_DTR] * _RSQ_HD, W[:, _DTR:]], axis=1)
        bd = jnp.concatenate([b[:_DTR] * _RSQ_HD, b[_DTR:]])
        return Wd, bd
    l0_Wqkv, l0_bqkv = scale_qkv(l0_Wqkv, l0_bqkv)
    l1_Wqkv, l1_bqkv = scale_qkv(l1_Wqkv, l1_bqkv)
    src_bm = jnp.transpose(src, (1, 0, 2))
    src_p = jnp.pad(src_bm, ((0, 0), (0, _TP - _T), (0, 0))
                    ).reshape(_B * _TP, 72)
    times_p = jnp.pad(jnp.transpose(times), ((0, 0), (0, _TP - _T))
                      ).reshape(_B * _TP, 1)
    len_i = lengths.astype(jnp.int32).reshape(_B, 1, 1)
    ts = jnp.asarray(1.0 / _TSCALES.astype(np.float64)
                     ).astype(jnp.float32).reshape(1, _NPE)

    def row(v):
        return v.reshape(1, -1)

    weights = [
        W_enc, row(b_enc), Wq, row(bq), Wk, row(bk), Wv, row(bv),
        Wskip, row(bskip),
        l0_Wqkv, row(l0_bqkv), l0_Wo, row(l0_bo), l0_W1, row(l0_b1),
        l0_W2, row(l0_b2), row(l0_ln1_g), row(l0_ln1_b), row(l0_ln2_g), row(l0_ln2_b),
        l1_Wqkv, row(l1_bqkv), l1_Wo, row(l1_bo), l1_W1, row(l1_b1),
        l1_W2, row(l1_b2), row(l1_ln1_g), row(l1_ln1_b), row(l1_ln2_g), row(l1_ln2_b),
        Wm1, row(bm1), Wm2, row(bm2),
    ]

    in_specs = [
        pl.BlockSpec((_R, 72), lambda b: (b, 0)),
        pl.BlockSpec((_R, 1), lambda b: (b, 0)),
        pl.BlockSpec((_BB, 1, 1), lambda b: (b, 0, 0), memory_space=pltpu.SMEM),
        _full2d(ts),
    ] + [_full2d(w) for w in weights]

    out_specs = [
        pl.BlockSpec((_BB, 2), lambda b: (b, 0)),
        pl.BlockSpec((_BB, _DINP, _DINP), lambda b: (b, 0, 0)),
    ]
    out_shape = [
        jax.ShapeDtypeStruct((_B, 2), f32),
        jax.ShapeDtypeStruct((_B, _DINP, _DINP), f32),
    ]

    logits, alpha = pl.pallas_call(
        _fwd_kernel,
        grid=(_B // _BB,),
        in_specs=in_specs,
        out_specs=out_specs,
        out_shape=out_shape,
        compiler_params=pltpu.CompilerParams(
            dimension_semantics=("parallel",)),
    )(src_p, times_p, len_i, ts, *weights)

    X = alpha.reshape(_B, _DINP * _DINP)
    dist = pl.pallas_call(
        _dist_kernel,
        out_shape=jax.ShapeDtypeStruct((1, 1), f32),
    )(X)
    return logits, dist[0, 0]


# speedup vs baseline: 1.1499x; 1.1499x over previous
import math

import numpy as np
import jax
import jax.numpy as jnp
from jax.experimental import pallas as pl
from jax.experimental.pallas import tpu as pltpu

_T = 215
_TP = 216
_B = 128
_BB = 8
_R = _BB * _TP
_DINP = 36
_DM = 144
_DTR = 180
_NH = 4
_HD = 45
_DPE = 36
_NPE = _DPE // 2
_DFIN = 108

_TSCALES = (float(_T) ** np.linspace(0.0, 1.0, _NPE)).astype(np.float32)

_RSQ_D = 1.0 / math.sqrt(float(_DM))
_RSQ_HD = 1.0 / math.sqrt(float(_HD))
_SQRT_DM = math.sqrt(float(_DM))


def _dot(a, b):
    return jnp.dot(a, b, preferred_element_type=jnp.float32)


def _dotb(a, b):
    return jnp.dot(a.astype(jnp.bfloat16), b.astype(jnp.bfloat16),
                   preferred_element_type=jnp.float32)


def _bdot_qk(q, k):
    return jax.lax.dot_general(
        q, k, (((2,), (2,)), ((0,), (0,))),
        preferred_element_type=jnp.float32)


def _bdot_av(a, v):
    return jax.lax.dot_general(
        a, v, (((2,), (1,)), ((0,), (0,))),
        preferred_element_type=jnp.float32)


def _layer_norm(x, g, b):
    mu = jnp.mean(x, axis=-1, keepdims=True)
    var = jnp.mean((x - mu) ** 2, axis=-1, keepdims=True)
    return (x - mu) * jax.lax.rsqrt(var + 1e-5) * g + b


def _enc_layer(h, keymask_rows, Wqkv, bqkv, Wo, bo, W1, b1, W2, b2,
               g1, be1, g2, be2):
    qkv = _dot(h, Wqkv) + bqkv
    outs = []
    for hh in range(_NH):
        qh = qkv[:, hh * _HD:(hh + 1) * _HD].reshape(_BB, _TP, _HD)
        kh = qkv[:, _DTR + hh * _HD:_DTR + (hh + 1) * _HD].reshape(_BB, _TP, _HD)
        vh = qkv[:, 2 * _DTR + hh * _HD:2 * _DTR + (hh + 1) * _HD].reshape(_BB, _TP, _HD)
        s = _bdot_qk(qh, kh).reshape(_R, _TP)
        s = jnp.where(keymask_rows, -1e9, s)
        smax = jnp.max(s, axis=1, keepdims=True)
        p = jnp.exp(s - smax)
        den = jnp.sum(p, axis=1, keepdims=True)
        pv = _bdot_av(p.reshape(_BB, _TP, _TP), vh).reshape(_R, _HD)
        outs.append(pv / den)
    o = jnp.concatenate(outs, axis=1)
    o = _dot(o, Wo) + bo
    h = _layer_norm(h + o, g1, be1)
    ff = jnp.maximum(_dot(h, W1) + b1, 0.0)
    ff = _dot(ff, W2) + b2
    return _layer_norm(h + ff, g2, be2)


def _fwd_kernel(src_ref, times_ref, len_ref, ts_ref,
                W_enc_ref, b_enc_ref,
                Wq_ref, bq_ref, Wk_ref, bk_ref, Wv_ref, bv_ref,
                Wskip_ref, bskip_ref,
                l0_Wqkv, l0_bqkv, l0_Wo, l0_bo, l0_W1, l0_b1, l0_W2, l0_b2,
                l0_g1, l0_be1, l0_g2, l0_be2,
                l1_Wqkv, l1_bqkv, l1_Wo, l1_bo, l1_W1, l1_b1, l1_W2, l1_b2,
                l1_g1, l1_be1, l1_g2, l1_be2,
                Wm1_ref, bm1_ref, Wm2_ref, bm2_ref,
                out_ref, alpha_ref):
    lens = jnp.concatenate(
        [jnp.broadcast_to(len_ref[j, 0, 0], (1, 1)) for j in range(_BB)],
        axis=0)

    xr = src_ref[:, :_DINP]
    x = (_dot(xr, W_enc_ref[...]) + b_enc_ref[...]) * _SQRT_DM

    sc = times_ref[...] * ts_ref[...]
    pe = jnp.concatenate([jnp.sin(sc), jnp.cos(sc)], axis=1)

    skip = _dot(x, Wskip_ref[...]) + bskip_ref[...]
    pieces = []
    for j in range(_BB):
        x36 = x[j * _TP:j * _TP + _DINP]
        q = _dot(x36, Wq_ref[...]) + bq_ref[...]
        k = _dot(x36, Wk_ref[...]) + bk_ref[...]
        v = _dot(x36, Wv_ref[...]) + bv_ref[...]
        s = _dot(q, k.T) * _RSQ_D
        smax = jnp.max(s, axis=1, keepdims=True)
        p = jnp.exp(s - smax)
        attn = p / (jnp.sum(p, axis=1, keepdims=True) + 1e-16)
        alpha_ref[j] = attn
        o_g = _dot(attn, v)
        pieces.append(skip[j * _TP:j * _TP + _DINP] + o_g)
        pieces.append(skip[j * _TP + _DINP:(j + 1) * _TP])
    outs = jnp.concatenate(pieces, axis=0)

    h = jnp.concatenate([outs, pe], axis=1)

    lane = jax.lax.broadcasted_iota(jnp.int32, (_R, _TP), 1)
    lens_rows = jnp.concatenate(
        [jnp.broadcast_to(lens[j, 0], (_TP, 1)) for j in range(_BB)], axis=0)
    keymask_rows = lane >= lens_rows

    h = _enc_layer(h, keymask_rows,
                   l0_Wqkv[...], l0_bqkv[...], l0_Wo[...], l0_bo[...],
                   l0_W1[...], l0_b1[...], l0_W2[...], l0_b2[...],
                   l0_g1[...], l0_be1[...], l0_g2[...], l0_be2[...])
    h = _enc_layer(h, keymask_rows,
                   l1_Wqkv[...], l1_bqkv[...], l1_Wo[...], l1_bo[...],
                   l1_W1[...], l1_b1[...], l1_W2[...], l1_b2[...],
                   l1_g1[...], l1_be1[...], l1_g2[...], l1_be2[...])

    lane2 = jax.lax.broadcasted_iota(jnp.int32, (_BB, _R), 1)
    rowbase = jax.lax.broadcasted_iota(jnp.int32, (_BB, _R), 0) * _TP
    t_local = lane2 - rowbase
    keep = ((t_local >= 0) & (t_local < lens)).astype(jnp.float32)
    agg = _dot(keep, h) / (lens.astype(jnp.float32) + 1.0)

    feat = agg[:, :_DFIN]
    hid = jnp.maximum(_dot(feat, Wm1_ref[...]) + bm1_ref[...], 0.0)
    out_ref[...] = _dot(hid, Wm2_ref[...]) + bm2_ref[...]


def _dist_kernel(x_ref, o_ref):
    X = x_ref[...]

    def body(i, acc):
        row = x_ref[pl.ds(i, 1), :]
        d = X - row
        ssq = jnp.sum(d * d, axis=1, keepdims=True)
        return acc + jnp.sum(jnp.sqrt(jnp.maximum(ssq, 1e-24)))

    tot = jax.lax.fori_loop(0, _B, body, jnp.float32(0.0))
    o_ref[...] = jnp.broadcast_to(tot / float(_B * _B), (1, 1))


def _full2d(a):
    return pl.BlockSpec(a.shape, lambda b: (0,) * a.ndim)


def kernel(src, static, times, lengths, adj, W_enc, b_enc, W_emb, b_emb,
           Wq, bq, Wk, bk, Wv, bv, Wskip, bskip,
           l0_Wqkv, l0_bqkv, l0_Wo, l0_bo, l0_W1, l0_b1, l0_W2, l0_b2,
           l0_ln1_g, l0_ln1_b, l0_ln2_g, l0_ln2_b,
           l1_Wqkv, l1_bqkv, l1_Wo, l1_bo, l1_W1, l1_b1, l1_W2, l1_b2,
           l1_ln1_g, l1_ln1_b, l1_ln2_g, l1_ln2_b,
           Wm1, bm1, Wm2, bm2):
    f32 = jnp.float32
    def scale_qkv(W, b):
        Wd = jnp.concatenate([W[:, :_DTR] * _RSQ_HD, W[:, _DTR:]], axis=1)
        bd = jnp.concatenate([b[:_DTR] * _RSQ_HD, b[_DTR:]])
        return Wd, bd
    l0_Wqkv, l0_bqkv = scale_qkv(l0_Wqkv, l0_bqkv)
    l1_Wqkv, l1_bqkv = scale_qkv(l1_Wqkv, l1_bqkv)
    src_bm = jnp.transpose(src, (1, 0, 2))
    src_p = jnp.pad(src_bm, ((0, 0), (0, _TP - _T), (0, 0))
                    ).reshape(_B * _TP, 72)
    times_p = jnp.pad(jnp.transpose(times), ((0, 0), (0, _TP - _T))
                      ).reshape(_B * _TP, 1)
    len_i = lengths.astype(jnp.int32).reshape(_B, 1, 1)
    ts = jnp.asarray(1.0 / _TSCALES.astype(np.float64)
                     ).astype(jnp.float32).reshape(1, _NPE)

    def row(v):
        return v.reshape(1, -1)

    weights = [
        W_enc, row(b_enc), Wq, row(bq), Wk, row(bk), Wv, row(bv),
        Wskip, row(bskip),
        l0_Wqkv, row(l0_bqkv), l0_Wo, row(l0_bo), l0_W1, row(l0_b1),
        l0_W2, row(l0_b2), row(l0_ln1_g), row(l0_ln1_b), row(l0_ln2_g), row(l0_ln2_b),
        l1_Wqkv, row(l1_bqkv), l1_Wo, row(l1_bo), l1_W1, row(l1_b1),
        l1_W2, row(l1_b2), row(l1_ln1_g), row(l1_ln1_b), row(l1_ln2_g), row(l1_ln2_b),
        Wm1, row(bm1), Wm2, row(bm2),
    ]

    in_specs = [
        pl.BlockSpec((_R, 72), lambda b: (b, 0)),
        pl.BlockSpec((_R, 1), lambda b: (b, 0)),
        pl.BlockSpec((_BB, 1, 1), lambda b: (b, 0, 0), memory_space=pltpu.SMEM),
        _full2d(ts),
    ] + [_full2d(w) for w in weights]

    out_specs = [
        pl.BlockSpec((_BB, 2), lambda b: (b, 0)),
        pl.BlockSpec((_BB, _DINP, _DINP), lambda b: (b, 0, 0)),
    ]
    out_shape = [
        jax.ShapeDtypeStruct((_B, 2), f32),
        jax.ShapeDtypeStruct((_B, _DINP, _DINP), f32),
    ]

    logits, alpha = pl.pallas_call(
        _fwd_kernel,
        grid=(_B // _BB,),
        in_specs=in_specs,
        out_specs=out_specs,
        out_shape=out_shape,
        compiler_params=pltpu.CompilerParams(
            dimension_semantics=("parallel",)),
    )(src_p, times_p, len_i, ts, *weights)

    X = alpha.reshape(_B, _DINP * _DINP)
    dist = pl.pallas_call(
        _dist_kernel,
        out_shape=jax.ShapeDtypeStruct((1, 1), f32),
    )(X)
    return logits, dist[0, 0]


# speedup vs baseline: 1.1773x; 1.0238x over previous
import math

import numpy as np
import jax
import jax.numpy as jnp
from jax.experimental import pallas as pl
from jax.experimental.pallas import tpu as pltpu

_T = 215
_TP = 216
_B = 128
_BB = 8
_R = _BB * _TP
_DINP = 36
_DM = 144
_DTR = 180
_NH = 4
_HD = 45
_DPE = 36
_NPE = _DPE // 2
_DFIN = 108

_TSCALES = (float(_T) ** np.linspace(0.0, 1.0, _NPE)).astype(np.float32)

_RSQ_D = 1.0 / math.sqrt(float(_DM))
_RSQ_HD = 1.0 / math.sqrt(float(_HD))
_SQRT_DM = math.sqrt(float(_DM))


def _dot(a, b):
    return jnp.dot(a, b, preferred_element_type=jnp.float32)


def _dotb(a, b):
    return jnp.dot(a.astype(jnp.bfloat16), b.astype(jnp.bfloat16),
                   preferred_element_type=jnp.float32)


def _bdot_qk(q, k):
    return jax.lax.dot_general(
        q, k, (((2,), (2,)), ((0,), (0,))),
        preferred_element_type=jnp.float32)


def _bdot_av(a, v):
    return jax.lax.dot_general(
        a, v, (((2,), (1,)), ((0,), (0,))),
        preferred_element_type=jnp.float32)


def _layer_norm(x, g, b):
    mu = jnp.mean(x, axis=-1, keepdims=True)
    var = jnp.mean((x - mu) ** 2, axis=-1, keepdims=True)
    return (x - mu) * jax.lax.rsqrt(var + 1e-5) * g + b


def _enc_layer(h, keymask_rows, Wqkv, bqkv, Wo, bo, W1, b1, W2, b2,
               g1, be1, g2, be2):
    qkv = _dot(h, Wqkv) + bqkv
    outs = []
    for hh in range(_NH):
        qh = qkv[:, hh * _HD:(hh + 1) * _HD].reshape(_BB, _TP, _HD)
        kh = qkv[:, _DTR + hh * _HD:_DTR + (hh + 1) * _HD].reshape(_BB, _TP, _HD)
        vh = qkv[:, 2 * _DTR + hh * _HD:2 * _DTR + (hh + 1) * _HD].reshape(_BB, _TP, _HD)
        s = _bdot_qk(qh, kh).reshape(_R, _TP)
        s = jnp.where(keymask_rows, -1e9, s)
        smax = jnp.max(s, axis=1, keepdims=True)
        p = jnp.exp(s - smax)
        den = jnp.sum(p, axis=1, keepdims=True)
        pv = _bdot_av(p.reshape(_BB, _TP, _TP), vh).reshape(_R, _HD)
        outs.append(pv / den)
    o = jnp.concatenate(outs, axis=1)
    o = _dot(o, Wo) + bo
    h = _layer_norm(h + o, g1, be1)
    ff = jnp.maximum(_dot(h, W1) + b1, 0.0)
    ff = _dot(ff, W2) + b2
    return _layer_norm(h + ff, g2, be2)


def _fwd_kernel(src_ref, times_ref, len_ref, ts_ref,
                W_enc_ref, b_enc_ref,
                Wq_ref, bq_ref, Wk_ref, bk_ref, Wv_ref, bv_ref,
                Wskip_ref, bskip_ref,
                l0_Wqkv, l0_bqkv, l0_Wo, l0_bo, l0_W1, l0_b1, l0_W2, l0_b2,
                l0_g1, l0_be1, l0_g2, l0_be2,
                l1_Wqkv, l1_bqkv, l1_Wo, l1_bo, l1_W1, l1_b1, l1_W2, l1_b2,
                l1_g1, l1_be1, l1_g2, l1_be2,
                Wm1_ref, bm1_ref, Wm2_ref, bm2_ref,
                out_ref, alpha_ref):
    lens = jnp.concatenate(
        [jnp.broadcast_to(len_ref[j, 0, 0], (1, 1)) for j in range(_BB)],
        axis=0)

    s3 = jnp.transpose(src_ref[...][:, :, :_DINP], (1, 0, 2))
    s3 = jnp.concatenate(
        [s3, jnp.zeros((_BB, _TP - _T, _DINP), jnp.float32)], axis=1)
    xr = s3.reshape(_R, _DINP)
    x = (_dot(xr, W_enc_ref[...]) + b_enc_ref[...]) * _SQRT_DM

    t3 = jnp.transpose(times_ref[...], (1, 0, 2))
    t3 = jnp.concatenate(
        [t3, jnp.zeros((_BB, _TP - _T, 1), jnp.float32)], axis=1)
    tcol = t3.reshape(_R, 1)

    sc = tcol * ts_ref[...]
    pe = jnp.concatenate([jnp.sin(sc), jnp.cos(sc)], axis=1)

    skip = _dot(x, Wskip_ref[...]) + bskip_ref[...]
    pieces = []
    for j in range(_BB):
        x36 = x[j * _TP:j * _TP + _DINP]
        q = _dot(x36, Wq_ref[...]) + bq_ref[...]
        k = _dot(x36, Wk_ref[...]) + bk_ref[...]
        v = _dot(x36, Wv_ref[...]) + bv_ref[...]
        s = _dot(q, k.T) * _RSQ_D
        smax = jnp.max(s, axis=1, keepdims=True)
        p = jnp.exp(s - smax)
        attn = p / (jnp.sum(p, axis=1, keepdims=True) + 1e-16)
        alpha_ref[j] = attn
        o_g = _dot(attn, v)
        pieces.append(skip[j * _TP:j * _TP + _DINP] + o_g)
        pieces.append(skip[j * _TP + _DINP:(j + 1) * _TP])
    outs = jnp.concatenate(pieces, axis=0)

    h = jnp.concatenate([outs, pe], axis=1)

    lane = jax.lax.broadcasted_iota(jnp.int32, (_R, _TP), 1)
    lens_rows = jnp.concatenate(
        [jnp.broadcast_to(lens[j, 0], (_TP, 1)) for j in range(_BB)], axis=0)
    keymask_rows = lane >= lens_rows

    h = _enc_layer(h, keymask_rows,
                   l0_Wqkv[...], l0_bqkv[...], l0_Wo[...], l0_bo[...],
                   l0_W1[...], l0_b1[...], l0_W2[...], l0_b2[...],
                   l0_g1[...], l0_be1[...], l0_g2[...], l0_be2[...])
    h = _enc_layer(h, keymask_rows,
                   l1_Wqkv[...], l1_bqkv[...], l1_Wo[...], l1_bo[...],
                   l1_W1[...], l1_b1[...], l1_W2[...], l1_b2[...],
                   l1_g1[...], l1_be1[...], l1_g2[...], l1_be2[...])

    lane2 = jax.lax.broadcasted_iota(jnp.int32, (_BB, _R), 1)
    rowbase = jax.lax.broadcasted_iota(jnp.int32, (_BB, _R), 0) * _TP
    t_local = lane2 - rowbase
    keep = ((t_local >= 0) & (t_local < lens)).astype(jnp.float32)
    agg = _dot(keep, h) / (lens.astype(jnp.float32) + 1.0)

    feat = agg[:, :_DFIN]
    hid = jnp.maximum(_dot(feat, Wm1_ref[...]) + bm1_ref[...], 0.0)
    out_ref[...] = _dot(hid, Wm2_ref[...]) + bm2_ref[...]


def _dist_kernel(x_ref, o_ref):
    X = x_ref[...]

    def body(i, acc):
        row = x_ref[pl.ds(i, 1), :]
        d = X - row
        ssq = jnp.sum(d * d, axis=1, keepdims=True)
        return acc + jnp.sum(jnp.sqrt(jnp.maximum(ssq, 1e-24)))

    tot = jax.lax.fori_loop(0, _B, body, jnp.float32(0.0))
    o_ref[...] = jnp.broadcast_to(tot / float(_B * _B), (1, 1))


def _full2d(a):
    return pl.BlockSpec(a.shape, lambda b: (0,) * a.ndim)


def kernel(src, static, times, lengths, adj, W_enc, b_enc, W_emb, b_emb,
           Wq, bq, Wk, bk, Wv, bv, Wskip, bskip,
           l0_Wqkv, l0_bqkv, l0_Wo, l0_bo, l0_W1, l0_b1, l0_W2, l0_b2,
           l0_ln1_g, l0_ln1_b, l0_ln2_g, l0_ln2_b,
           l1_Wqkv, l1_bqkv, l1_Wo, l1_bo, l1_W1, l1_b1, l1_W2, l1_b2,
           l1_ln1_g, l1_ln1_b, l1_ln2_g, l1_ln2_b,
           Wm1, bm1, Wm2, bm2):
    f32 = jnp.float32
    def scale_qkv(W, b):
        Wd = jnp.concatenate([W[:, :_DTR] * _RSQ_HD, W[:, _DTR:]], axis=1)
        bd = jnp.concatenate([b[:_DTR] * _RSQ_HD, b[_DTR:]])
        return Wd, bd
    l0_Wqkv, l0_bqkv = scale_qkv(l0_Wqkv, l0_bqkv)
    l1_Wqkv, l1_bqkv = scale_qkv(l1_Wqkv, l1_bqkv)
    src_p = src
    times_p = times.reshape(_T, _B, 1)
    len_i = lengths.astype(jnp.int32).reshape(_B, 1, 1)
    ts = jnp.asarray(1.0 / _TSCALES.astype(np.float64)
                     ).astype(jnp.float32).reshape(1, _NPE)

    def row(v):
        return v.reshape(1, -1)

    weights = [
        W_enc, row(b_enc), Wq, row(bq), Wk, row(bk), Wv, row(bv),
        Wskip, row(bskip),
        l0_Wqkv, row(l0_bqkv), l0_Wo, row(l0_bo), l0_W1, row(l0_b1),
        l0_W2, row(l0_b2), row(l0_ln1_g), row(l0_ln1_b), row(l0_ln2_g), row(l0_ln2_b),
        l1_Wqkv, row(l1_bqkv), l1_Wo, row(l1_bo), l1_W1, row(l1_b1),
        l1_W2, row(l1_b2), row(l1_ln1_g), row(l1_ln1_b), row(l1_ln2_g), row(l1_ln2_b),
        Wm1, row(bm1), Wm2, row(bm2),
    ]

    in_specs = [
        pl.BlockSpec((_T, _BB, 72), lambda b: (0, b, 0)),
        pl.BlockSpec((_T, _BB, 1), lambda b: (0, b, 0)),
        pl.BlockSpec((_BB, 1, 1), lambda b: (b, 0, 0), memory_space=pltpu.SMEM),
        _full2d(ts),
    ] + [_full2d(w) for w in weights]

    out_specs = [
        pl.BlockSpec((_BB, 2), lambda b: (b, 0)),
        pl.BlockSpec((_BB, _DINP, _DINP), lambda b: (b, 0, 0)),
    ]
    out_shape = [
        jax.ShapeDtypeStruct((_B, 2), f32),
        jax.ShapeDtypeStruct((_B, _DINP, _DINP), f32),
    ]

    logits, alpha = pl.pallas_call(
        _fwd_kernel,
        grid=(_B // _BB,),
        in_specs=in_specs,
        out_specs=out_specs,
        out_shape=out_shape,
        compiler_params=pltpu.CompilerParams(
            dimension_semantics=("parallel",)),
    )(src_p, times_p, len_i, ts, *weights)

    X = alpha.reshape(_B, _DINP * _DINP)
    dist = pl.pallas_call(
        _dist_kernel,
        out_shape=jax.ShapeDtypeStruct((1, 1), f32),
    )(X)
    return logits, dist[0, 0]


# speedup vs baseline: 1.2794x; 1.0867x over previous
import math

import numpy as np
import jax
import jax.numpy as jnp
from jax.experimental import pallas as pl
from jax.experimental.pallas import tpu as pltpu

_T = 215
_TP = 216
_B = 128
_BB = 8
_R = _BB * _TP
_DINP = 36
_DM = 144
_DTR = 180
_NH = 4
_HD = 45
_DPE = 36
_NPE = _DPE // 2
_DFIN = 108

_TSCALES = (float(_T) ** np.linspace(0.0, 1.0, _NPE)).astype(np.float32)

_RSQ_D = 1.0 / math.sqrt(float(_DM))
_RSQ_HD = 1.0 / math.sqrt(float(_HD))
_SQRT_DM = math.sqrt(float(_DM))


def _dot(a, b):
    return jnp.dot(a, b, preferred_element_type=jnp.float32)


def _dotb(a, b):
    return jnp.dot(a.astype(jnp.bfloat16), b.astype(jnp.bfloat16),
                   preferred_element_type=jnp.float32)


def _bdot_qk(q, k):
    return jax.lax.dot_general(
        q, k, (((2,), (2,)), ((0,), (0,))),
        preferred_element_type=jnp.float32)


def _bdot_av(a, v):
    return jax.lax.dot_general(
        a, v, (((2,), (1,)), ((0,), (0,))),
        preferred_element_type=jnp.float32)


def _layer_norm(x, g, b):
    mu = jnp.mean(x, axis=-1, keepdims=True)
    var = jnp.mean((x - mu) ** 2, axis=-1, keepdims=True)
    return (x - mu) * jax.lax.rsqrt(var + 1e-5) * g + b


def _enc_layer(h, keymask_rows, Wqkv, bqkv, Wo, bo, W1, b1, W2, b2,
               g1, be1, g2, be2):
    qkv = _dot(h, Wqkv) + bqkv
    outs = []
    for hh in range(_NH):
        qh = qkv[:, hh * _HD:(hh + 1) * _HD].reshape(_BB, _TP, _HD)
        kh = qkv[:, _DTR + hh * _HD:_DTR + (hh + 1) * _HD].reshape(_BB, _TP, _HD)
        vh = qkv[:, 2 * _DTR + hh * _HD:2 * _DTR + (hh + 1) * _HD].reshape(_BB, _TP, _HD)
        s = _bdot_qk(qh, kh).reshape(_R, _TP)
        s = jnp.where(keymask_rows, -1e9, s)
        smax = jnp.max(s, axis=1, keepdims=True)
        p = jnp.exp(s - smax)
        den = jnp.sum(p, axis=1, keepdims=True)
        pv = _bdot_av(p.reshape(_BB, _TP, _TP), vh).reshape(_R, _HD)
        outs.append(pv / den)
    o = jnp.concatenate(outs, axis=1)
    o = _dot(o, Wo) + bo
    h = _layer_norm(h + o, g1, be1)
    ff = jnp.maximum(_dot(h, W1) + b1, 0.0)
    ff = _dot(ff, W2) + b2
    return _layer_norm(h + ff, g2, be2)


def _fwd_kernel(src_ref, times_ref, len_ref, ts_ref,
                W_enc_ref, b_enc_ref,
                Wq_ref, bq_ref, Wk_ref, bk_ref, Wv_ref, bv_ref,
                Wskip_ref, bskip_ref,
                l0_Wqkv, l0_bqkv, l0_Wo, l0_bo, l0_W1, l0_b1, l0_W2, l0_b2,
                l0_g1, l0_be1, l0_g2, l0_be2,
                l1_Wqkv, l1_bqkv, l1_Wo, l1_bo, l1_W1, l1_b1, l1_W2, l1_b2,
                l1_g1, l1_be1, l1_g2, l1_be2,
                Wm1_ref, bm1_ref, Wm2_ref, bm2_ref,
                out_ref, alpha_ref):
    lens = jnp.concatenate(
        [jnp.broadcast_to(len_ref[j, 0, 0], (1, 1)) for j in range(_BB)],
        axis=0)

    s3 = jnp.transpose(src_ref[...][:, :, :_DINP], (1, 0, 2))
    s3 = jnp.concatenate(
        [s3, jnp.zeros((_BB, _TP - _T, _DINP), jnp.float32)], axis=1)
    xr = s3.reshape(_R, _DINP)
    x = (_dot(xr, W_enc_ref[...]) + b_enc_ref[...]) * _SQRT_DM

    t3 = jnp.transpose(times_ref[...], (1, 0, 2))
    t3 = jnp.concatenate(
        [t3, jnp.zeros((_BB, _TP - _T, 1), jnp.float32)], axis=1)
    tcol = t3.reshape(_R, 1)

    sc = tcol * ts_ref[...]
    pe = jnp.concatenate([jnp.sin(sc), jnp.cos(sc)], axis=1)

    skip = _dot(x, Wskip_ref[...]) + bskip_ref[...]
    pieces = []
    for j in range(_BB):
        x36 = x[j * _TP:j * _TP + _DINP]
        q = _dot(x36, Wq_ref[...]) + bq_ref[...]
        k = _dot(x36, Wk_ref[...]) + bk_ref[...]
        v = _dot(x36, Wv_ref[...]) + bv_ref[...]
        s = _dot(q, k.T) * _RSQ_D
        smax = jnp.max(s, axis=1, keepdims=True)
        p = jnp.exp(s - smax)
        attn = p / (jnp.sum(p, axis=1, keepdims=True) + 1e-16)
        alpha_ref[j] = attn
        o_g = _dot(attn, v)
        pieces.append(skip[j * _TP:j * _TP + _DINP] + o_g)
        pieces.append(skip[j * _TP + _DINP:(j + 1) * _TP])
    outs = jnp.concatenate(pieces, axis=0)

    h = jnp.concatenate([outs, pe], axis=1)

    lane = jax.lax.broadcasted_iota(jnp.int32, (_R, _TP), 1)
    lens_rows = jnp.concatenate(
        [jnp.broadcast_to(lens[j, 0], (_TP, 1)) for j in range(_BB)], axis=0)
    keymask_rows = lane >= lens_rows

    h = _enc_layer(h, keymask_rows,
                   l0_Wqkv[...], l0_bqkv[...], l0_Wo[...], l0_bo[...],
                   l0_W1[...], l0_b1[...], l0_W2[...], l0_b2[...],
                   l0_g1[...], l0_be1[...], l0_g2[...], l0_be2[...])
    h = _enc_layer(h, keymask_rows,
                   l1_Wqkv[...], l1_bqkv[...], l1_Wo[...], l1_bo[...],
                   l1_W1[...], l1_b1[...], l1_W2[...], l1_b2[...],
                   l1_g1[...], l1_be1[...], l1_g2[...], l1_be2[...])

    lane2 = jax.lax.broadcasted_iota(jnp.int32, (_BB, _R), 1)
    rowbase = jax.lax.broadcasted_iota(jnp.int32, (_BB, _R), 0) * _TP
    t_local = lane2 - rowbase
    keep = ((t_local >= 0) & (t_local < lens)).astype(jnp.float32)
    agg = _dot(keep, h) / (lens.astype(jnp.float32) + 1.0)

    feat = agg[:, :_DFIN]
    hid = jnp.maximum(_dot(feat, Wm1_ref[...]) + bm1_ref[...], 0.0)
    out_ref[...] = _dot(hid, Wm2_ref[...]) + bm2_ref[...]


def _dist_kernel(x_ref, o_ref):
    X = x_ref[...]
    G = jnp.dot(X, X.T, preferred_element_type=jnp.float32)
    eye = (jax.lax.broadcasted_iota(jnp.int32, (_B, _B), 0) ==
           jax.lax.broadcasted_iota(jnp.int32, (_B, _B), 1))
    Gd = jnp.where(eye, G, 0.0)
    n_col = jnp.sum(Gd, axis=1, keepdims=True)
    n_row = jnp.sum(Gd, axis=0, keepdims=True)
    d2 = n_col + n_row - 2.0 * G
    dmat = jnp.sqrt(jnp.maximum(d2, 1e-24))
    o_ref[...] = jnp.broadcast_to(jnp.sum(dmat) / float(_B * _B), (1, 1))


def _full2d(a):
    return pl.BlockSpec(a.shape, lambda b: (0,) * a.ndim)


def kernel(src, static, times, lengths, adj, W_enc, b_enc, W_emb, b_emb,
           Wq, bq, Wk, bk, Wv, bv, Wskip, bskip,
           l0_Wqkv, l0_bqkv, l0_Wo, l0_bo, l0_W1, l0_b1, l0_W2, l0_b2,
           l0_ln1_g, l0_ln1_b, l0_ln2_g, l0_ln2_b,
           l1_Wqkv, l1_bqkv, l1_Wo, l1_bo, l1_W1, l1_b1, l1_W2, l1_b2,
           l1_ln1_g, l1_ln1_b, l1_ln2_g, l1_ln2_b,
           Wm1, bm1, Wm2, bm2):
    f32 = jnp.float32
    def scale_qkv(W, b):
        Wd = jnp.concatenate([W[:, :_DTR] * _RSQ_HD, W[:, _DTR:]], axis=1)
        bd = jnp.concatenate([b[:_DTR] * _RSQ_HD, b[_DTR:]])
        return Wd, bd
    l0_Wqkv, l0_bqkv = scale_qkv(l0_Wqkv, l0_bqkv)
    l1_Wqkv, l1_bqkv = scale_qkv(l1_Wqkv, l1_bqkv)
    src_p = src
    times_p = times.reshape(_T, _B, 1)
    len_i = lengths.astype(jnp.int32).reshape(_B, 1, 1)
    ts = jnp.asarray(1.0 / _TSCALES.astype(np.float64)
                     ).astype(jnp.float32).reshape(1, _NPE)

    def row(v):
        return v.reshape(1, -1)

    weights = [
        W_enc, row(b_enc), Wq, row(bq), Wk, row(bk), Wv, row(bv),
        Wskip, row(bskip),
        l0_Wqkv, row(l0_bqkv), l0_Wo, row(l0_bo), l0_W1, row(l0_b1),
        l0_W2, row(l0_b2), row(l0_ln1_g), row(l0_ln1_b), row(l0_ln2_g), row(l0_ln2_b),
        l1_Wqkv, row(l1_bqkv), l1_Wo, row(l1_bo), l1_W1, row(l1_b1),
        l1_W2, row(l1_b2), row(l1_ln1_g), row(l1_ln1_b), row(l1_ln2_g), row(l1_ln2_b),
        Wm1, row(bm1), Wm2, row(bm2),
    ]

    in_specs = [
        pl.BlockSpec((_T, _BB, 72), lambda b: (0, b, 0)),
        pl.BlockSpec((_T, _BB, 1), lambda b: (0, b, 0)),
        pl.BlockSpec((_BB, 1, 1), lambda b: (b, 0, 0), memory_space=pltpu.SMEM),
        _full2d(ts),
    ] + [_full2d(w) for w in weights]

    out_specs = [
        pl.BlockSpec((_BB, 2), lambda b: (b, 0)),
        pl.BlockSpec((_BB, _DINP, _DINP), lambda b: (b, 0, 0)),
    ]
    out_shape = [
        jax.ShapeDtypeStruct((_B, 2), f32),
        jax.ShapeDtypeStruct((_B, _DINP, _DINP), f32),
    ]

    logits, alpha = pl.pallas_call(
        _fwd_kernel,
        grid=(_B // _BB,),
        in_specs=in_specs,
        out_specs=out_specs,
        out_shape=out_shape,
        compiler_params=pltpu.CompilerParams(
            dimension_semantics=("parallel",)),
    )(src_p, times_p, len_i, ts, *weights)

    X = alpha.reshape(_B, _DINP * _DINP)
    dist = pl.pallas_call(
        _dist_kernel,
        out_shape=jax.ShapeDtypeStruct((1, 1), f32),
    )(X)
    return logits, dist[0, 0]


# speedup vs baseline: 1.3073x; 1.0218x over previous
import math

import numpy as np
import jax
import jax.numpy as jnp
from jax.experimental import pallas as pl
from jax.experimental.pallas import tpu as pltpu

_T = 215
_TP = 216
_B = 128
_BB = 8
_R = _BB * _TP
_DINP = 36
_DM = 144
_DTR = 180
_NH = 4
_HD = 45
_DPE = 36
_NPE = _DPE // 2
_DFIN = 108

_TSCALES = (float(_T) ** np.linspace(0.0, 1.0, _NPE)).astype(np.float32)

_RSQ_D = 1.0 / math.sqrt(float(_DM))
_RSQ_HD = 1.0 / math.sqrt(float(_HD))
_SQRT_DM = math.sqrt(float(_DM))


def _dot(a, b):
    return jnp.dot(a, b, preferred_element_type=jnp.float32)


def _dotb(a, b):
    return jnp.dot(a.astype(jnp.bfloat16), b.astype(jnp.bfloat16),
                   preferred_element_type=jnp.float32)


def _bdot_qk(q, k):
    return jax.lax.dot_general(
        q, k, (((2,), (2,)), ((0,), (0,))),
        preferred_element_type=jnp.float32)


def _bdot_av(a, v):
    return jax.lax.dot_general(
        a, v, (((2,), (1,)), ((0,), (0,))),
        preferred_element_type=jnp.float32)


def _layer_norm(x, g, b):
    mu = jnp.mean(x, axis=-1, keepdims=True)
    var = jnp.mean(x * x, axis=-1, keepdims=True) - mu * mu
    return (x - mu) * jax.lax.rsqrt(var + 1e-5) * g + b


def _enc_layer(h, keymask_rows, Wqkv, bqkv, Wo, bo, W1, b1, W2, b2,
               g1, be1, g2, be2):
    qkv = _dot(h, Wqkv) + bqkv
    outs = []
    for hh in range(_NH):
        qh = qkv[:, hh * _HD:(hh + 1) * _HD].reshape(_BB, _TP, _HD)
        kh = qkv[:, _DTR + hh * _HD:_DTR + (hh + 1) * _HD].reshape(_BB, _TP, _HD)
        vh = qkv[:, 2 * _DTR + hh * _HD:2 * _DTR + (hh + 1) * _HD].reshape(_BB, _TP, _HD)
        s = _bdot_qk(qh, kh).reshape(_R, _TP)
        s = jnp.where(keymask_rows, -1e9, s)
        smax = jnp.max(s, axis=1, keepdims=True)
        p = jnp.exp(s - smax)
        den = jnp.sum(p, axis=1, keepdims=True)
        pv = _bdot_av(p.reshape(_BB, _TP, _TP), vh).reshape(_R, _HD)
        outs.append(pv / den)
    o = jnp.concatenate(outs, axis=1)
    o = _dot(o, Wo) + bo
    h = _layer_norm(h + o, g1, be1)
    ff = jnp.maximum(_dot(h, W1) + b1, 0.0)
    ff = _dot(ff, W2) + b2
    return _layer_norm(h + ff, g2, be2)


def _fwd_kernel(src_ref, times_ref, len_ref, ts_ref,
                W_enc_ref, b_enc_ref,
                Wq_ref, bq_ref, Wk_ref, bk_ref, Wv_ref, bv_ref,
                Wskip_ref, bskip_ref,
                l0_Wqkv, l0_bqkv, l0_Wo, l0_bo, l0_W1, l0_b1, l0_W2, l0_b2,
                l0_g1, l0_be1, l0_g2, l0_be2,
                l1_Wqkv, l1_bqkv, l1_Wo, l1_bo, l1_W1, l1_b1, l1_W2, l1_b2,
                l1_g1, l1_be1, l1_g2, l1_be2,
                Wm1_ref, bm1_ref, Wm2_ref, bm2_ref,
                out_ref, alpha_ref):
    lens = jnp.concatenate(
        [jnp.broadcast_to(len_ref[j, 0, 0], (1, 1)) for j in range(_BB)],
        axis=0)

    s3 = jnp.transpose(src_ref[...][:, :, :_DINP], (1, 0, 2))
    s3 = jnp.concatenate(
        [s3, jnp.zeros((_BB, _TP - _T, _DINP), jnp.float32)], axis=1)
    xr = s3.reshape(_R, _DINP)
    x = (_dot(xr, W_enc_ref[...]) + b_enc_ref[...]) * _SQRT_DM

    t3 = jnp.transpose(times_ref[...], (1, 0, 2))
    t3 = jnp.concatenate(
        [t3, jnp.zeros((_BB, _TP - _T, 1), jnp.float32)], axis=1)
    tcol = t3.reshape(_R, 1)

    sc = tcol * ts_ref[...]

    skip = _dot(x, Wskip_ref[...]) + bskip_ref[...]
    pieces = []
    for j in range(_BB):
        x36 = x[j * _TP:j * _TP + _DINP]
        q = _dot(x36, Wq_ref[...]) + bq_ref[...]
        k = _dot(x36, Wk_ref[...]) + bk_ref[...]
        v = _dot(x36, Wv_ref[...]) + bv_ref[...]
        s = _dot(q, k.T) * _RSQ_D
        smax = jnp.max(s, axis=1, keepdims=True)
        p = jnp.exp(s - smax)
        attn = p / (jnp.sum(p, axis=1, keepdims=True) + 1e-16)
        alpha_ref[j] = attn
        o_g = _dot(attn, v)
        pieces.append(skip[j * _TP:j * _TP + _DINP] + o_g)
        pieces.append(skip[j * _TP + _DINP:(j + 1) * _TP])
    outs = jnp.concatenate(pieces, axis=0)

    h = jnp.concatenate([outs, jnp.sin(sc), jnp.cos(sc)], axis=1)

    lane = jax.lax.broadcasted_iota(jnp.int32, (_R, _TP), 1)
    lens_rows = jnp.concatenate(
        [jnp.broadcast_to(lens[j, 0], (_TP, 1)) for j in range(_BB)], axis=0)
    keymask_rows = lane >= lens_rows

    h = _enc_layer(h, keymask_rows,
                   l0_Wqkv[...], l0_bqkv[...], l0_Wo[...], l0_bo[...],
                   l0_W1[...], l0_b1[...], l0_W2[...], l0_b2[...],
                   l0_g1[...], l0_be1[...], l0_g2[...], l0_be2[...])
    h = _enc_layer(h, keymask_rows,
                   l1_Wqkv[...], l1_bqkv[...], l1_Wo[...], l1_bo[...],
                   l1_W1[...], l1_b1[...], l1_W2[...], l1_b2[...],
                   l1_g1[...], l1_be1[...], l1_g2[...], l1_be2[...])

    lane2 = jax.lax.broadcasted_iota(jnp.int32, (_BB, _R), 1)
    rowbase = jax.lax.broadcasted_iota(jnp.int32, (_BB, _R), 0) * _TP
    t_local = lane2 - rowbase
    keep = ((t_local >= 0) & (t_local < lens)).astype(jnp.float32)
    agg = _dot(keep, h) / (lens.astype(jnp.float32) + 1.0)

    feat = agg[:, :_DFIN]
    hid = jnp.maximum(_dot(feat, Wm1_ref[...]) + bm1_ref[...], 0.0)
    out_ref[...] = _dot(hid, Wm2_ref[...]) + bm2_ref[...]


def _dist_kernel(x_ref, o_ref):
    X = x_ref[...]
    G = jnp.dot(X, X.T, preferred_element_type=jnp.float32)
    eye = (jax.lax.broadcasted_iota(jnp.int32, (_B, _B), 0) ==
           jax.lax.broadcasted_iota(jnp.int32, (_B, _B), 1))
    Gd = jnp.where(eye, G, 0.0)
    n_col = jnp.sum(Gd, axis=1, keepdims=True)
    n_row = jnp.sum(Gd, axis=0, keepdims=True)
    d2 = n_col + n_row - 2.0 * G
    dmat = jnp.sqrt(jnp.maximum(d2, 1e-24))
    o_ref[...] = jnp.broadcast_to(jnp.sum(dmat) / float(_B * _B), (1, 1))


def _full2d(a):
    return pl.BlockSpec(a.shape, lambda b: (0,) * a.ndim)


def kernel(src, static, times, lengths, adj, W_enc, b_enc, W_emb, b_emb,
           Wq, bq, Wk, bk, Wv, bv, Wskip, bskip,
           l0_Wqkv, l0_bqkv, l0_Wo, l0_bo, l0_W1, l0_b1, l0_W2, l0_b2,
           l0_ln1_g, l0_ln1_b, l0_ln2_g, l0_ln2_b,
           l1_Wqkv, l1_bqkv, l1_Wo, l1_bo, l1_W1, l1_b1, l1_W2, l1_b2,
           l1_ln1_g, l1_ln1_b, l1_ln2_g, l1_ln2_b,
           Wm1, bm1, Wm2, bm2):
    f32 = jnp.float32
    qscale = jnp.asarray(
        np.concatenate([np.full(_DTR, _RSQ_HD, np.float32),
                        np.ones(2 * _DTR, np.float32)]))
    l0_Wqkv, l0_bqkv = l0_Wqkv * qscale, l0_bqkv * qscale
    l1_Wqkv, l1_bqkv = l1_Wqkv * qscale, l1_bqkv * qscale
    src_p = src
    times_p = times.reshape(_T, _B, 1)
    len_i = lengths.astype(jnp.int32).reshape(_B, 1, 1)
    ts = jnp.asarray(1.0 / _TSCALES.astype(np.float64)
                     ).astype(jnp.float32).reshape(1, _NPE)

    def row(v):
        return v.reshape(1, -1)

    weights = [
        W_enc, row(b_enc), Wq, row(bq), Wk, row(bk), Wv, row(bv),
        Wskip, row(bskip),
        l0_Wqkv, row(l0_bqkv), l0_Wo, row(l0_bo), l0_W1, row(l0_b1),
        l0_W2, row(l0_b2), row(l0_ln1_g), row(l0_ln1_b), row(l0_ln2_g), row(l0_ln2_b),
        l1_Wqkv, row(l1_bqkv), l1_Wo, row(l1_bo), l1_W1, row(l1_b1),
        l1_W2, row(l1_b2), row(l1_ln1_g), row(l1_ln1_b), row(l1_ln2_g), row(l1_ln2_b),
        Wm1, row(bm1), Wm2, row(bm2),
    ]

    in_specs = [
        pl.BlockSpec((_T, _BB, 72), lambda b: (0, b, 0)),
        pl.BlockSpec((_T, _BB, 1), lambda b: (0, b, 0)),
        pl.BlockSpec((_BB, 1, 1), lambda b: (b, 0, 0), memory_space=pltpu.SMEM),
        _full2d(ts),
    ] + [_full2d(w) for w in weights]

    out_specs = [
        pl.BlockSpec((_BB, 2), lambda b: (b, 0)),
        pl.BlockSpec((_BB, _DINP, _DINP), lambda b: (b, 0, 0)),
    ]
    out_shape = [
        jax.ShapeDtypeStruct((_B, 2), f32),
        jax.ShapeDtypeStruct((_B, _DINP, _DINP), f32),
    ]

    logits, alpha = pl.pallas_call(
        _fwd_kernel,
        grid=(_B // _BB,),
        in_specs=in_specs,
        out_specs=out_specs,
        out_shape=out_shape,
        compiler_params=pltpu.CompilerParams(
            dimension_semantics=("parallel",)),
    )(src_p, times_p, len_i, ts, *weights)

    X = alpha.reshape(_B, _DINP * _DINP)
    dist = pl.pallas_call(
        _dist_kernel,
        out_shape=jax.ShapeDtypeStruct((1, 1), f32),
    )(X)
    return logits, dist[0, 0]


# speedup vs baseline: 1.3083x; 1.0008x over previous
import math

import numpy as np
import jax
import jax.numpy as jnp
from jax.experimental import pallas as pl
from jax.experimental.pallas import tpu as pltpu

_T = 215
_TP = 216
_B = 128
_BB = 8
_R = _BB * _TP
_DINP = 36
_DM = 144
_DTR = 180
_NH = 4
_HD = 45
_DPE = 36
_NPE = _DPE // 2
_DFIN = 108

_TSCALES = (float(_T) ** np.linspace(0.0, 1.0, _NPE)).astype(np.float32)

_RSQ_D = 1.0 / math.sqrt(float(_DM))
_RSQ_HD = 1.0 / math.sqrt(float(_HD))
_SQRT_DM = math.sqrt(float(_DM))


def _dot(a, b):
    return jnp.dot(a, b, preferred_element_type=jnp.float32)


def _bdot_qk(q, k):
    return jax.lax.dot_general(
        q, k, (((2,), (2,)), ((0,), (0,))),
        preferred_element_type=jnp.float32)


def _bdot_av(a, v):
    return jax.lax.dot_general(
        a, v, (((2,), (1,)), ((0,), (0,))),
        preferred_element_type=jnp.float32)


def _layer_norm(x, g, b):
    mu = jnp.mean(x, axis=-1, keepdims=True)
    var = jnp.mean(x * x, axis=-1, keepdims=True) - mu * mu
    return (x - mu) * jax.lax.rsqrt(var + 1e-5) * g + b


def _enc_layer(h, keymask_rows, Wqkv, bqkv, Wo, bo, W1, b1, W2, b2,
               g1, be1, g2, be2):
    qkv = _dot(h, Wqkv) + bqkv
    outs = []
    for hh in range(_NH):
        qh = qkv[:, hh * _HD:(hh + 1) * _HD].reshape(_BB, _TP, _HD)
        kh = qkv[:, _DTR + hh * _HD:_DTR + (hh + 1) * _HD].reshape(_BB, _TP, _HD)
        vh = qkv[:, 2 * _DTR + hh * _HD:2 * _DTR + (hh + 1) * _HD].reshape(_BB, _TP, _HD)
        s = _bdot_qk(qh, kh).reshape(_R, _TP)
        s = jnp.where(keymask_rows, -1e9, s)
        smax = jnp.max(s, axis=1, keepdims=True)
        p = jnp.exp(s - smax)
        den = jnp.sum(p, axis=1, keepdims=True)
        pv = _bdot_av(p.reshape(_BB, _TP, _TP), vh).reshape(_R, _HD)
        outs.append(pv / den)
    o = jnp.concatenate(outs, axis=1)
    o = _dot(o, Wo) + bo
    h = _layer_norm(h + o, g1, be1)
    ff = jnp.maximum(_dot(h, W1) + b1, 0.0)
    ff = _dot(ff, W2) + b2
    return _layer_norm(h + ff, g2, be2)


def _fwd_kernel(src_ref, times_ref, len_ref, ts_ref,
                W_enc_ref, b_enc_ref,
                Wq_ref, bq_ref, Wk_ref, bk_ref, Wv_ref, bv_ref,
                Wskip_ref, bskip_ref,
                l0_Wqkv, l0_bqkv, l0_Wo, l0_bo, l0_W1, l0_b1, l0_W2, l0_b2,
                l0_g1, l0_be1, l0_g2, l0_be2,
                l1_Wqkv, l1_bqkv, l1_Wo, l1_bo, l1_W1, l1_b1, l1_W2, l1_b2,
                l1_g1, l1_be1, l1_g2, l1_be2,
                Wm1_ref, bm1_ref, Wm2_ref, bm2_ref,
                out_ref, alpha_ref):
    lens = jnp.concatenate(
        [jnp.broadcast_to(len_ref[j, 0, 0], (1, 1)) for j in range(_BB)],
        axis=0)

    s3 = jnp.transpose(src_ref[...][:, :, :_DINP], (1, 0, 2))
    s3 = jnp.concatenate(
        [s3, jnp.zeros((_BB, _TP - _T, _DINP), jnp.float32)], axis=1)
    xr = s3.reshape(_R, _DINP)
    x = (_dot(xr, W_enc_ref[...]) + b_enc_ref[...]) * _SQRT_DM

    t3 = jnp.transpose(times_ref[...], (1, 0, 2))
    t3 = jnp.concatenate(
        [t3, jnp.zeros((_BB, _TP - _T, 1), jnp.float32)], axis=1)
    tcol = t3.reshape(_R, 1)

    sc = tcol * ts_ref[...]

    skip = _dot(x, Wskip_ref[...]) + bskip_ref[...]
    pieces = []
    for j in range(_BB):
        x36 = x[j * _TP:j * _TP + _DINP]
        q = _dot(x36, Wq_ref[...]) + bq_ref[...]
        k = _dot(x36, Wk_ref[...]) + bk_ref[...]
        v = _dot(x36, Wv_ref[...]) + bv_ref[...]
        s = _dot(q, k.T) * _RSQ_D
        smax = jnp.max(s, axis=1, keepdims=True)
        p = jnp.exp(s - smax)
        attn = p / (jnp.sum(p, axis=1, keepdims=True) + 1e-16)
        alpha_ref[j] = attn
        o_g = _dot(attn, v)
        pieces.append(skip[j * _TP:j * _TP + _DINP] + o_g)
        pieces.append(skip[j * _TP + _DINP:(j + 1) * _TP])
    outs = jnp.concatenate(pieces, axis=0)

    h = jnp.concatenate([outs, jnp.sin(sc), jnp.cos(sc)], axis=1)

    lane = jax.lax.broadcasted_iota(jnp.int32, (_R, _TP), 1)
    lens_rows = jnp.concatenate(
        [jnp.broadcast_to(lens[j, 0], (_TP, 1)) for j in range(_BB)], axis=0)
    keymask_rows = lane >= lens_rows

    h = _enc_layer(h, keymask_rows,
                   l0_Wqkv[...], l0_bqkv[...], l0_Wo[...], l0_bo[...],
                   l0_W1[...], l0_b1[...], l0_W2[...], l0_b2[...],
                   l0_g1[...], l0_be1[...], l0_g2[...], l0_be2[...])
    h = _enc_layer(h, keymask_rows,
                   l1_Wqkv[...], l1_bqkv[...], l1_Wo[...], l1_bo[...],
                   l1_W1[...], l1_b1[...], l1_W2[...], l1_b2[...],
                   l1_g1[...], l1_be1[...], l1_g2[...], l1_be2[...])

    lane2 = jax.lax.broadcasted_iota(jnp.int32, (_BB, _R), 1)
    rowbase = jax.lax.broadcasted_iota(jnp.int32, (_BB, _R), 0) * _TP
    t_local = lane2 - rowbase
    keep = ((t_local >= 0) & (t_local < lens)).astype(jnp.float32)
    agg = _dot(keep, h) / (lens.astype(jnp.float32) + 1.0)

    feat = agg[:, :_DFIN]
    hid = jnp.maximum(_dot(feat, Wm1_ref[...]) + bm1_ref[...], 0.0)
    out_ref[...] = _dot(hid, Wm2_ref[...]) + bm2_ref[...]


def _dist_kernel(x_ref, o_ref):
    X = x_ref[...]
    G = jnp.dot(X, X.T, preferred_element_type=jnp.float32)
    eye = (jax.lax.broadcasted_iota(jnp.int32, (_B, _B), 0) ==
           jax.lax.broadcasted_iota(jnp.int32, (_B, _B), 1))
    Gd = jnp.where(eye, G, 0.0)
    n_col = jnp.sum(Gd, axis=1, keepdims=True)
    n_row = jnp.sum(Gd, axis=0, keepdims=True)
    d2 = n_col + n_row - 2.0 * G
    dmat = jnp.sqrt(jnp.maximum(d2, 1e-24))
    o_ref[...] = jnp.broadcast_to(jnp.sum(dmat) / float(_B * _B), (1, 1))


def _full2d(a):
    return pl.BlockSpec(a.shape, lambda b: (0,) * a.ndim)


def kernel(src, static, times, lengths, adj, W_enc, b_enc, W_emb, b_emb,
           Wq, bq, Wk, bk, Wv, bv, Wskip, bskip,
           l0_Wqkv, l0_bqkv, l0_Wo, l0_bo, l0_W1, l0_b1, l0_W2, l0_b2,
           l0_ln1_g, l0_ln1_b, l0_ln2_g, l0_ln2_b,
           l1_Wqkv, l1_bqkv, l1_Wo, l1_bo, l1_W1, l1_b1, l1_W2, l1_b2,
           l1_ln1_g, l1_ln1_b, l1_ln2_g, l1_ln2_b,
           Wm1, bm1, Wm2, bm2):
    f32 = jnp.float32
    qscale = jnp.asarray(
        np.concatenate([np.full(_DTR, _RSQ_HD, np.float32),
                        np.ones(2 * _DTR, np.float32)]))
    l0_Wqkv, l0_bqkv = l0_Wqkv * qscale, l0_bqkv * qscale
    l1_Wqkv, l1_bqkv = l1_Wqkv * qscale, l1_bqkv * qscale
    src_p = src
    times_p = times.reshape(_T, _B, 1)
    len_i = lengths.astype(jnp.int32).reshape(_B, 1, 1)
    ts = jnp.asarray(1.0 / _TSCALES.astype(np.float64)
                     ).astype(jnp.float32).reshape(1, _NPE)

    def row(v):
        return v.reshape(1, -1)

    weights = [
        W_enc, row(b_enc), Wq, row(bq), Wk, row(bk), Wv, row(bv),
        Wskip, row(bskip),
        l0_Wqkv, row(l0_bqkv), l0_Wo, row(l0_bo), l0_W1, row(l0_b1),
        l0_W2, row(l0_b2), row(l0_ln1_g), row(l0_ln1_b), row(l0_ln2_g), row(l0_ln2_b),
        l1_Wqkv, row(l1_bqkv), l1_Wo, row(l1_bo), l1_W1, row(l1_b1),
        l1_W2, row(l1_b2), row(l1_ln1_g), row(l1_ln1_b), row(l1_ln2_g), row(l1_ln2_b),
        Wm1, row(bm1), Wm2, row(bm2),
    ]

    in_specs = [
        pl.BlockSpec((_T, _BB, 72), lambda b: (0, b, 0)),
        pl.BlockSpec((_T, _BB, 1), lambda b: (0, b, 0)),
        pl.BlockSpec((_BB, 1, 1), lambda b: (b, 0, 0), memory_space=pltpu.SMEM),
        _full2d(ts),
    ] + [_full2d(w) for w in weights]

    out_specs = [
        pl.BlockSpec((_BB, 2), lambda b: (b, 0)),
        pl.BlockSpec((_BB, _DINP, _DINP), lambda b: (b, 0, 0)),
    ]
    out_shape = [
        jax.ShapeDtypeStruct((_B, 2), f32),
        jax.ShapeDtypeStruct((_B, _DINP, _DINP), f32),
    ]

    logits, alpha = pl.pallas_call(
        _fwd_kernel,
        grid=(_B // _BB,),
        in_specs=in_specs,
        out_specs=out_specs,
        out_shape=out_shape,
        compiler_params=pltpu.CompilerParams(
            dimension_semantics=("parallel",)),
    )(src_p, times_p, len_i, ts, *weights)

    X = alpha.reshape(_B, _DINP * _DINP)
    dist = pl.pallas_call(
        _dist_kernel,
        out_shape=jax.ShapeDtypeStruct((1, 1), f32),
    )(X)
    return logits, dist[0, 0]
